# baseline (speedup 1.0000x reference)
import functools

import jax
import jax.numpy as jnp
from jax import lax
from jax.experimental import pallas as pl
from jax.experimental.pallas import tpu as pltpu
from jax.experimental.pallas import tpu_sc as plsc

B, S, F = 2, 8192, 2048
T = 256
FB = 128
NV = FB // 16
G = S // T
NC, NS = 2, 16
NFBLK = F // FB


def _compute_block(buf, cs):
    def sbody(s, cs):
        out = []
        for j in range(NV):
            c = cs[j] + buf[s, j * 16:(j + 1) * 16]
            buf[s, j * 16:(j + 1) * 16] = c
            out.append(c)
        return tuple(out)

    return lax.fori_loop(0, T, sbody, cs)


NBUF = 3


def _scan_body(x_hbm, o_hbm, buf0, buf1, buf2, ld0, ld1, ld2, st0, st1, st2):
    wid = lax.axis_index("s") * NC + lax.axis_index("c")
    b = wid // NFBLK
    f0 = (wid % NFBLK) * FB
    bufs = (buf0, buf1, buf2)
    lds = (ld0, ld1, ld2)
    sts = (st0, st1, st2)

    def load(g, k):
        pltpu.make_async_copy(
            x_hbm.at[b, pl.ds(g * T, T), pl.ds(f0, FB)], bufs[k], lds[k]
        ).start()

    def store_start(g, k):
        pltpu.make_async_copy(
            bufs[k], o_hbm.at[b, pl.ds(g * T, T), pl.ds(f0, FB)], sts[k]
        ).start()

    def store_wait(g, k):
        pltpu.make_async_copy(
            bufs[k], o_hbm.at[b, pl.ds(g * T, T), pl.ds(f0, FB)], sts[k]
        ).wait()

    load(0, 0)
    load(1, 1)
    czero = jnp.zeros((16,), jnp.float32)

    def outer(i, cs):
        for k in range(NBUF):
            g = NBUF * i + k
            nk = (k + 2) % NBUF

            @pl.when((g >= 1) & (g + 2 < G))
            def _():
                store_wait(g - 1, nk)

            @pl.when(g + 2 < G)
            def _():
                load(g + 2, nk)

            pltpu.make_async_copy(
                x_hbm.at[b, pl.ds(g * T, T), pl.ds(f0, FB)], bufs[k], lds[k]
            ).wait()
            cs = _compute_block(bufs[k], cs)
            store_start(g, k)
        return cs

    cs = lax.fori_loop(0, G // NBUF, outer, (czero,) * NV)
    for g in range((G // NBUF) * NBUF, G):
        k = g % NBUF
        pltpu.make_async_copy(
            x_hbm.at[b, pl.ds(g * T, T), pl.ds(f0, FB)], bufs[k], lds[k]
        ).wait()
        cs = _compute_block(bufs[k], cs)
        store_start(g, k)
    for g in range(G - NBUF, G):
        store_wait(g, g % NBUF)


def kernel(x):

    mesh = plsc.VectorSubcoreMesh(core_axis_name="c", subcore_axis_name="s")

    scan = functools.partial(
        pl.kernel,
        mesh=mesh,
        out_type=jax.ShapeDtypeStruct((B, S, F), jnp.float32),
        scratch_types=(
            [pltpu.VMEM((T, FB), jnp.float32)] * NBUF
            + [pltpu.SemaphoreType.DMA] * (2 * NBUF)
        ),
    )(_scan_body)

    return scan(x)
